# baseline (speedup 1.0000x reference)
import jax
import jax.numpy as jnp
from jax.experimental import pallas as pl
from jax.experimental.pallas import tpu as pltpu

_EPS = 1e-12


def _gcn_body(x_ref, w_ref, adj_ref, o1_ref, o2_ref, sup_ref):
    @pl.when(pl.program_id(0) == 0)
    def _():
        n = x_ref.shape[0]
        chunk = 1000
        for j in range(n // chunk):
            sup_ref[pl.ds(j * chunk, chunk), :] = jax.lax.dot_general(
                x_ref[pl.ds(j * chunk, chunk), :], w_ref[...],
                (((1,), (0,)), ((), ())),
                preferred_element_type=jnp.float32,
                precision=jax.lax.Precision.HIGHEST)

    acc = jnp.dot(adj_ref[...], sup_ref[...],
                  preferred_element_type=jnp.float32)
    d = o1_ref.shape[1]
    h1 = acc[:, :d]
    h2 = acc[:, d:]
    n1 = jnp.sqrt(jnp.sum(h1 * h1, axis=1, keepdims=True))
    n2 = jnp.sqrt(jnp.sum(h2 * h2, axis=1, keepdims=True))
    o1_ref[...] = h1 / jnp.maximum(n1, _EPS)
    o2_ref[...] = h2 / jnp.maximum(n2, _EPS)


def kernel(x, adj, W1, W2):
    n, d_in = x.shape
    d_out = W1.shape[1]
    wcat = jnp.concatenate([W1, W2], axis=1)
    bm = 480

    out1, out2 = pl.pallas_call(
        _gcn_body,
        grid=(pl.cdiv(n, bm),),
        in_specs=[
            pl.BlockSpec((n, d_in), lambda i: (0, 0)),
            pl.BlockSpec((d_in, 2 * d_out), lambda i: (0, 0)),
            pl.BlockSpec((bm, n), lambda i: (i, 0)),
        ],
        out_specs=[
            pl.BlockSpec((bm, d_out), lambda i: (i, 0)),
            pl.BlockSpec((bm, d_out), lambda i: (i, 0)),
        ],
        out_shape=[
            jax.ShapeDtypeStruct((n, d_out), jnp.float32),
            jax.ShapeDtypeStruct((n, d_out), jnp.float32),
        ],
        scratch_shapes=[pltpu.VMEM((n, 2 * d_out), jnp.float32)],
        compiler_params=pltpu.CompilerParams(
            vmem_limit_bytes=100 * 1024 * 1024),
    )(x, wcat, adj)
    return (out1, out2)


# speedup vs baseline: 1.0508x; 1.0508x over previous
import jax
import jax.numpy as jnp
from jax.experimental import pallas as pl
from jax.experimental.pallas import tpu as pltpu

_EPS = 1e-12


def _gcn_body(x_ref, w_ref, adj_ref, o1_ref, o2_ref, sup_ref):
    @pl.when(pl.program_id(0) == 0)
    def _():
        n = x_ref.shape[0]
        chunk = 1000
        for j in range(n // chunk):
            sup_ref[pl.ds(j * chunk, chunk), :] = jax.lax.dot_general(
                x_ref[pl.ds(j * chunk, chunk), :], w_ref[...],
                (((1,), (0,)), ((), ())),
                preferred_element_type=jnp.float32)

    acc = jnp.dot(adj_ref[...], sup_ref[...],
                  preferred_element_type=jnp.float32)
    d = o1_ref.shape[1]
    h1 = acc[:, :d]
    h2 = acc[:, d:]
    n1 = jnp.sqrt(jnp.sum(h1 * h1, axis=1, keepdims=True))
    n2 = jnp.sqrt(jnp.sum(h2 * h2, axis=1, keepdims=True))
    o1_ref[...] = h1 / jnp.maximum(n1, _EPS)
    o2_ref[...] = h2 / jnp.maximum(n2, _EPS)


def kernel(x, adj, W1, W2):
    n, d_in = x.shape
    d_out = W1.shape[1]
    wcat = jnp.concatenate([W1, W2], axis=1)
    bm = 400

    out1, out2 = pl.pallas_call(
        _gcn_body,
        grid=(pl.cdiv(n, bm),),
        in_specs=[
            pl.BlockSpec((n, d_in), lambda i: (0, 0)),
            pl.BlockSpec((d_in, 2 * d_out), lambda i: (0, 0)),
            pl.BlockSpec((bm, n), lambda i: (i, 0)),
        ],
        out_specs=[
            pl.BlockSpec((bm, d_out), lambda i: (i, 0)),
            pl.BlockSpec((bm, d_out), lambda i: (i, 0)),
        ],
        out_shape=[
            jax.ShapeDtypeStruct((n, d_out), jnp.float32),
            jax.ShapeDtypeStruct((n, d_out), jnp.float32),
        ],
        scratch_shapes=[pltpu.VMEM((n, 2 * d_out), jnp.float32)],
        compiler_params=pltpu.CompilerParams(
            vmem_limit_bytes=100 * 1024 * 1024),
    )(x, wcat, adj)
    return (out1, out2)
